# baseline (speedup 1.0000x reference)
import math

import jax
import jax.numpy as jnp
import numpy as np
from jax.experimental import pallas as pl

DIM = 1024
SCALE = math.sqrt(DIM)
LO = 256


def _make_tables(seq):
    d = np.arange(DIM)
    freq = np.exp(-(2 * (d // 2)).astype(np.float64) * (math.log(10000.0) / DIM))
    even = (d % 2) == 0

    n_hi = seq // LO
    hi_angle = (LO * np.arange(n_hi, dtype=np.float64))[:, None] * freq[None, :]
    p_hi = np.where(even[None, :], np.sin(hi_angle), np.cos(hi_angle))
    q_hi = np.where(even[None, :], np.cos(hi_angle), -np.sin(hi_angle))

    lo_angle = np.arange(LO, dtype=np.float64)[:, None] * freq[None, :]
    c_lo = np.broadcast_to(np.cos(lo_angle)[:, None, :], (LO, 4, DIM))
    s_lo = np.broadcast_to(np.sin(lo_angle)[:, None, :], (LO, 4, DIM))

    return (
        p_hi.astype(np.float32)[:, None, :],
        q_hi.astype(np.float32)[:, None, :],
        np.ascontiguousarray(c_lo.astype(np.float32)),
        np.ascontiguousarray(s_lo.astype(np.float32)),
    )


_TABLES = _make_tables(4096)


def _block_kernel(emb_ref, p_ref, q_ref, cl_ref, sl_ref, out_ref):
    p = p_ref[0][None]
    q = q_ref[0][None]
    out_ref[...] = emb_ref[...] * SCALE + (p * cl_ref[...] + q * sl_ref[...])


def kernel(emb):
    seq, feat, dim = emb.shape
    grid = (seq // LO,)
    return pl.pallas_call(
        _block_kernel,
        grid=grid,
        in_specs=[
            pl.BlockSpec((LO, feat, dim), lambda i: (i, 0, 0)),
            pl.BlockSpec((1, 1, dim), lambda i: (i, 0, 0)),
            pl.BlockSpec((1, 1, dim), lambda i: (i, 0, 0)),
            pl.BlockSpec((LO, feat, dim), lambda i: (0, 0, 0)),
            pl.BlockSpec((LO, feat, dim), lambda i: (0, 0, 0)),
        ],
        out_specs=pl.BlockSpec((LO, feat, dim), lambda i: (i, 0, 0)),
        out_shape=jax.ShapeDtypeStruct((seq, feat, dim), emb.dtype),
    )(emb, *_TABLES)


# speedup vs baseline: 1.0734x; 1.0734x over previous
import math

import jax
import jax.numpy as jnp
import numpy as np
from jax.experimental import pallas as pl

DIM = 1024
SCALE = math.sqrt(DIM)
LO = 512


def _make_tables(seq):
    d = np.arange(DIM)
    freq = np.exp(-(2 * (d // 2)).astype(np.float64) * (math.log(10000.0) / DIM))
    even = (d % 2) == 0

    n_hi = seq // LO
    hi_angle = (LO * np.arange(n_hi, dtype=np.float64))[:, None] * freq[None, :]
    p_hi = np.where(even[None, :], np.sin(hi_angle), np.cos(hi_angle))
    q_hi = np.where(even[None, :], np.cos(hi_angle), -np.sin(hi_angle))

    lo_angle = np.arange(LO, dtype=np.float64)[:, None] * freq[None, :]
    c_lo = np.cos(lo_angle)
    s_lo = np.sin(lo_angle)

    return (
        p_hi.astype(np.float32)[:, None, :],
        q_hi.astype(np.float32)[:, None, :],
        c_lo.astype(np.float32),
        s_lo.astype(np.float32),
    )


_TABLES = _make_tables(4096)


def _block_kernel(emb_ref, p_ref, q_ref, cl_ref, sl_ref, out_ref):
    pe = p_ref[0] * cl_ref[...] + q_ref[0] * sl_ref[...]
    out_ref[...] = emb_ref[...] * SCALE + pe[:, None, :]


def kernel(emb):
    seq, feat, dim = emb.shape
    grid = (seq // LO,)
    return pl.pallas_call(
        _block_kernel,
        grid=grid,
        in_specs=[
            pl.BlockSpec((LO, feat, dim), lambda i: (i, 0, 0)),
            pl.BlockSpec((1, 1, dim), lambda i: (i, 0, 0)),
            pl.BlockSpec((1, 1, dim), lambda i: (i, 0, 0)),
            pl.BlockSpec((LO, dim), lambda i: (0, 0)),
            pl.BlockSpec((LO, dim), lambda i: (0, 0)),
        ],
        out_specs=pl.BlockSpec((LO, feat, dim), lambda i: (i, 0, 0)),
        out_shape=jax.ShapeDtypeStruct((seq, feat, dim), emb.dtype),
    )(emb, *_TABLES)
